# baseline (speedup 1.0000x reference)
import functools
import math

import jax
import jax.numpy as jnp
from jax import lax
from jax.experimental import pallas as pl
from jax.experimental.pallas import tpu as pltpu

NUM = 768
K = 192
INNER = 384
BOUND = 1.125
LO = math.log10(0.05)
HI = math.log10(1.0) - (math.log10(1.0) - math.log10(0.05)) / NUM
STEP = (HI - LO) / (NUM - 1)
LN10 = math.log(10.0)
NEG_BIG = -3.0e38

CAND_COLS = tuple(c for c in range(256) if c % 4 != 0)

R_BLOCK = 256


def _tc_ranks_body(o_ref, d_ref, ranks_ref):
    ci = lax.broadcasted_iota(jnp.int32, (1, NUM), 1)
    c = ci.astype(jnp.float32)
    zlog = LO + STEP * c
    z = jnp.exp(zlog * LN10)
    ox = o_ref[:, 0:1]
    oy = o_ref[:, 1:2]
    oz = o_ref[:, 2:3]
    dx = d_ref[:, 0:1]
    dy = d_ref[:, 1:2]
    dz = d_ref[:, 2:3]
    x = ox + z * dx
    y = oy + z * dy
    w = oz + z * dz
    m2 = x * x + y * y + w * w
    rinv = lax.rsqrt(m2)
    s2 = jnp.where(m2 <= 1.0, m2, (2.0 - rinv) * (2.0 - rinv)) * (
        1.0 / (BOUND * BOUND))
    sigma = 25.0 * jnp.exp(-2.0 * s2)
    a = sigma * STEP
    alpha = 1.0 - jnp.exp(-a)
    l1a = jnp.log(1.0 - alpha)
    q = l1a
    sh = 1
    while sh < NUM:
        q = q + jnp.concatenate(
            [jnp.zeros((q.shape[0], sh), jnp.float32), q[:, :NUM - sh]], axis=1)
        sh *= 2
    q = q - l1a
    t = jnp.log(alpha) + q
    unmasked = (ci % 4) != 0
    t = jnp.where(jnp.isfinite(t), t, NEG_BIG)
    t_cmp = jnp.where(unmasked, t, NEG_BIG - 0.0)
    in_a = unmasked & (ci < 256)

    ranks_ref[...] = jnp.full((ranks_ref.shape[0], 256), 1000, jnp.int32)
    for cc in CAND_COLS:
        tc = t[:, cc:cc + 1]
        gt = t_cmp > tc
        tie = (t_cmp == tc) & in_a & (ci < cc)
        cnt = jnp.sum((gt | tie).astype(jnp.float32), axis=1, keepdims=True)
        ranks_ref[:, cc:cc + 1] = cnt.astype(jnp.int32)


def _tc_ranks(rays_o, rays_d):
    n = rays_o.shape[0]
    grid = n // R_BLOCK
    return pl.pallas_call(
        _tc_ranks_body,
        grid=(grid,),
        in_specs=[
            pl.BlockSpec((R_BLOCK, 3), lambda i: (i, 0)),
            pl.BlockSpec((R_BLOCK, 3), lambda i: (i, 0)),
        ],
        out_specs=pl.BlockSpec((R_BLOCK, 256), lambda i: (i, 0)),
        out_shape=jax.ShapeDtypeStruct((n, 256), jnp.int32),
    )(rays_o, rays_d)


def _sc_compact(ranks):
    import numpy as np
    from jax.experimental.pallas import tpu_sc as plsc

    n = ranks.shape[0]
    info = plsc.get_sparse_core_info()
    nw = info.num_cores * info.num_subcores
    nc = info.num_cores
    rays_per_w = n // nw
    ch = 16
    n_chunks = rays_per_w // ch
    mesh = plsc.VectorSubcoreMesh(core_axis_name="c", subcore_axis_name="s")

    @functools.partial(
        pl.kernel,
        mesh=mesh,
        compiler_params=pltpu.CompilerParams(needs_layout_passes=False),
        out_type=jax.ShapeDtypeStruct((n, INNER), jnp.float32),
        scratch_types=[
            pltpu.VMEM((ch, 256), jnp.int32),
            pltpu.VMEM((ch, INNER), jnp.float32),
            pltpu.VMEM((576,), jnp.int32),
            pltpu.VMEM((576,), jnp.int32),
        ],
    )
    def sc_kernel(ranks_hbm, out_hbm, in_v, out_v, sel_v, g_v):
        wid = lax.axis_index("s") * nc + lax.axis_index("c")
        i16 = lax.iota(jnp.int32, 16)
        zero16 = i16 * 0
        ones16 = zero16 + 1

        def chunk_body(ci, _):
            base = wid * rays_per_w + ci * ch
            pltpu.sync_copy(ranks_hbm.at[pl.ds(base, ch)], in_v)

            def ray_body(r, __):
                rfull = zero16 + r
                for s in range(36):
                    sel_v[pl.ds(16 * s, 16)] = zero16
                for gch in range(16):
                    rv = in_v[r, pl.ds(16 * gch, 16)]
                    plsc.store_scatter(sel_v, [rv], ones16, mask=rv < 576)
                carry = jnp.int32(0)
                for s in range(36):
                    v = sel_v[pl.ds(16 * s, 16)]
                    cs = plsc.cumsum(v) + carry
                    g_v[pl.ds(16 * s, 16)] = cs
                    carry = carry + jnp.sum(v)
                    jj = i16 + (16 * s)
                    q3 = jj // 3
                    colj = q3 + jj + 1
                    slot = q3 + cs
                    val = LO + STEP * colj.astype(jnp.float32)
                    plsc.store_scatter(out_v, [rfull, slot], val, mask=v > 0)
                for mm in range(12):
                    m = i16 + (16 * mm)
                    gidx = jnp.maximum(3 * m - 1, 0)
                    gv = plsc.load_gather(g_v, [gidx])
                    if mm == 0:
                        gv = jnp.where(i16 == 0, 0, gv)
                    slot = m + gv
                    val = LO + STEP * 4.0 * m.astype(jnp.float32)
                    plsc.store_scatter(out_v, [rfull, slot], val)
                return __

            lax.fori_loop(0, ch, ray_body, 0)
            pltpu.sync_copy(out_v, out_hbm.at[pl.ds(base, ch)])
            return _

        lax.fori_loop(0, n_chunks, chunk_body, 0)

    return sc_kernel(ranks)


def _backhalf_jnp(ranks):
    n = ranks.shape[0]
    r = ranks[:, jnp.asarray(CAND_COLS)]
    jidx = jnp.arange(576)
    sel = jnp.any(r[:, :, None] == jidx[None, None, :], axis=1)
    g = jnp.cumsum(sel.astype(jnp.int32), axis=1)
    colj = 4 * (jidx // 3) + (jidx % 3) + 1
    slot_u = jidx // 3 + g
    val_u = (LO + STEP * colj).astype(jnp.float32)
    midx = jnp.arange(K)
    gprev = jnp.where(midx == 0, 0, jnp.take_along_axis(
        g, jnp.maximum(3 * midx - 1, 0)[None, :].repeat(n, 0), axis=1))
    slot_m = midx[None, :] + gprev
    val_m = (LO + STEP * 4.0 * midx).astype(jnp.float32)
    out = jnp.zeros((n, INNER), jnp.float32)
    out = out.at[jnp.arange(n)[:, None], slot_m].set(
        jnp.broadcast_to(val_m[None, :], (n, K)))
    out = out.at[jnp.arange(n)[:, None], jnp.where(sel, slot_u, INNER)].set(
        jnp.broadcast_to(val_u[None, :], (n, 576)), mode="drop")
    return out


def kernel(rays_o, rays_d, n_samples):
    ranks = _tc_ranks(rays_o, rays_d)
    return _sc_compact(ranks)


# speedup vs baseline: 1.2376x; 1.2376x over previous
import functools
import math

import jax
import jax.numpy as jnp
from jax import lax
from jax.experimental import pallas as pl
from jax.experimental.pallas import tpu as pltpu

NUM = 768
K = 192
INNER = 384
BOUND = 1.125
LO = math.log10(0.05)
HI = math.log10(1.0) - (math.log10(1.0) - math.log10(0.05)) / NUM
STEP = (HI - LO) / (NUM - 1)
LN10 = math.log(10.0)
NEG_BIG = -3.0e38

CAND_COLS = tuple(c for c in range(256) if c % 4 != 0)

R_BLOCK = 64


def _tc_ranks_body(o_ref, d_ref, ranks_ref):
    ci = lax.broadcasted_iota(jnp.int32, (1, NUM), 1)
    c = ci.astype(jnp.float32)
    zlog = LO + STEP * c
    z = jnp.exp(zlog * LN10)
    ox = o_ref[:, 0:1]
    oy = o_ref[:, 1:2]
    oz = o_ref[:, 2:3]
    dx = d_ref[:, 0:1]
    dy = d_ref[:, 1:2]
    dz = d_ref[:, 2:3]
    x = ox + z * dx
    y = oy + z * dy
    w = oz + z * dz
    m2 = x * x + y * y + w * w
    rinv = lax.rsqrt(m2)
    s2 = jnp.where(m2 <= 1.0, m2, (2.0 - rinv) * (2.0 - rinv)) * (
        1.0 / (BOUND * BOUND))
    sigma = 25.0 * jnp.exp(-2.0 * s2)
    a = sigma * STEP
    alpha = 1.0 - jnp.exp(-a)
    l1a = jnp.log(1.0 - alpha)
    q = l1a
    sh = 1
    while sh < NUM:
        q = q + jnp.concatenate(
            [jnp.zeros((q.shape[0], sh), jnp.float32), q[:, :NUM - sh]], axis=1)
        sh *= 2
    q = q - l1a
    t = jnp.log(alpha) + q
    unmasked = (ci % 4) != 0
    t = jnp.where(jnp.isfinite(t), t, NEG_BIG)
    t_cmp = jnp.where(unmasked, t, NEG_BIG - 0.0)
    in_a = unmasked & (ci < 256)

    ranks_ref[...] = jnp.full((ranks_ref.shape[0], 256), 1000, jnp.int32)
    ones_col = jnp.ones((NUM, 1), jnp.float32)
    for cc in CAND_COLS:
        tc = t[:, cc:cc + 1]
        gt = t_cmp > tc
        tie = (t_cmp == tc) & in_a & (ci < cc)
        pred = jnp.where(gt | tie, 1.0, 0.0)
        cnt = jax.lax.dot_general(
            pred, ones_col, (((1,), (0,)), ((), ())),
            preferred_element_type=jnp.float32)
        ranks_ref[:, cc:cc + 1] = cnt.astype(jnp.int32)


def _tc_ranks(rays_o, rays_d):
    n = rays_o.shape[0]
    grid = n // R_BLOCK
    return pl.pallas_call(
        _tc_ranks_body,
        grid=(grid,),
        in_specs=[
            pl.BlockSpec((R_BLOCK, 3), lambda i: (i, 0)),
            pl.BlockSpec((R_BLOCK, 3), lambda i: (i, 0)),
        ],
        out_specs=pl.BlockSpec((R_BLOCK, 256), lambda i: (i, 0)),
        out_shape=jax.ShapeDtypeStruct((n, 256), jnp.int32),
    )(rays_o, rays_d)


def _sc_compact(ranks):
    import numpy as np
    from jax.experimental.pallas import tpu_sc as plsc

    n = ranks.shape[0]
    info = plsc.get_sparse_core_info()
    nw = info.num_cores * info.num_subcores
    nc = info.num_cores
    rays_per_w = n // nw
    ch = 16
    n_chunks = rays_per_w // ch
    mesh = plsc.VectorSubcoreMesh(core_axis_name="c", subcore_axis_name="s")

    @functools.partial(
        pl.kernel,
        mesh=mesh,
        compiler_params=pltpu.CompilerParams(needs_layout_passes=False),
        out_type=jax.ShapeDtypeStruct((n, INNER), jnp.float32),
        scratch_types=[
            pltpu.VMEM((ch, 256), jnp.int32),
            pltpu.VMEM((ch, INNER), jnp.float32),
            pltpu.VMEM((576,), jnp.int32),
            pltpu.VMEM((576,), jnp.int32),
        ],
    )
    def sc_kernel(ranks_hbm, out_hbm, in_v, out_v, sel_v, g_v):
        wid = lax.axis_index("s") * nc + lax.axis_index("c")
        i16 = lax.iota(jnp.int32, 16)
        zero16 = i16 * 0
        ones16 = zero16 + 1

        def chunk_body(ci, _):
            base = wid * rays_per_w + ci * ch
            pltpu.sync_copy(ranks_hbm.at[pl.ds(base, ch)], in_v)

            def ray_body(r, __):
                rfull = zero16 + r
                for s in range(36):
                    sel_v[pl.ds(16 * s, 16)] = zero16
                for gch in range(16):
                    rv = in_v[r, pl.ds(16 * gch, 16)]
                    plsc.store_scatter(sel_v, [rv], ones16, mask=rv < 576)
                carry = jnp.int32(0)
                for s in range(36):
                    v = sel_v[pl.ds(16 * s, 16)]
                    cs = plsc.cumsum(v) + carry
                    g_v[pl.ds(16 * s, 16)] = cs
                    carry = carry + jnp.sum(v)
                    jj = i16 + (16 * s)
                    q3 = jj // 3
                    colj = q3 + jj + 1
                    slot = q3 + cs
                    val = LO + STEP * colj.astype(jnp.float32)
                    plsc.store_scatter(out_v, [rfull, slot], val, mask=v > 0)
                for mm in range(12):
                    m = i16 + (16 * mm)
                    gidx = jnp.maximum(3 * m - 1, 0)
                    gv = plsc.load_gather(g_v, [gidx])
                    if mm == 0:
                        gv = jnp.where(i16 == 0, 0, gv)
                    slot = m + gv
                    val = LO + STEP * 4.0 * m.astype(jnp.float32)
                    plsc.store_scatter(out_v, [rfull, slot], val)
                return __

            lax.fori_loop(0, ch, ray_body, 0)
            pltpu.sync_copy(out_v, out_hbm.at[pl.ds(base, ch)])
            return _

        lax.fori_loop(0, n_chunks, chunk_body, 0)

    return sc_kernel(ranks)


def _backhalf_jnp(ranks):
    n = ranks.shape[0]
    r = ranks[:, jnp.asarray(CAND_COLS)]
    jidx = jnp.arange(576)
    sel = jnp.any(r[:, :, None] == jidx[None, None, :], axis=1)
    g = jnp.cumsum(sel.astype(jnp.int32), axis=1)
    colj = 4 * (jidx // 3) + (jidx % 3) + 1
    slot_u = jidx // 3 + g
    val_u = (LO + STEP * colj).astype(jnp.float32)
    midx = jnp.arange(K)
    gprev = jnp.where(midx == 0, 0, jnp.take_along_axis(
        g, jnp.maximum(3 * midx - 1, 0)[None, :].repeat(n, 0), axis=1))
    slot_m = midx[None, :] + gprev
    val_m = (LO + STEP * 4.0 * midx).astype(jnp.float32)
    out = jnp.zeros((n, INNER), jnp.float32)
    out = out.at[jnp.arange(n)[:, None], slot_m].set(
        jnp.broadcast_to(val_m[None, :], (n, K)))
    out = out.at[jnp.arange(n)[:, None], jnp.where(sel, slot_u, INNER)].set(
        jnp.broadcast_to(val_u[None, :], (n, 576)), mode="drop")
    return out


def kernel(rays_o, rays_d, n_samples):
    ranks = _tc_ranks(rays_o, rays_d)
    return _sc_compact(ranks)


# speedup vs baseline: 1.5996x; 1.2924x over previous
import functools
import math

import jax
import jax.numpy as jnp
from jax import lax
from jax.experimental import pallas as pl
from jax.experimental.pallas import tpu as pltpu

NUM = 768
K = 192
INNER = 384
BOUND = 1.125
LO = math.log10(0.05)
HI = math.log10(1.0) - (math.log10(1.0) - math.log10(0.05)) / NUM
STEP = (HI - LO) / (NUM - 1)
LN10 = math.log(10.0)
NEG_BIG = -3.0e38

CAND_COLS = tuple(c for c in range(256) if c % 4 != 0)

R_BLOCK = 128


def _tc_ranks_body(o_ref, d_ref, ranks_ref):
    ci = lax.broadcasted_iota(jnp.int32, (1, NUM), 1)
    c = ci.astype(jnp.float32)
    zlog = LO + STEP * c
    z = jnp.exp(zlog * LN10)
    ox = o_ref[:, 0:1]
    oy = o_ref[:, 1:2]
    oz = o_ref[:, 2:3]
    dx = d_ref[:, 0:1]
    dy = d_ref[:, 1:2]
    dz = d_ref[:, 2:3]
    x = ox + z * dx
    y = oy + z * dy
    w = oz + z * dz
    m2 = x * x + y * y + w * w
    rinv = lax.rsqrt(m2)
    s2 = jnp.where(m2 <= 1.0, m2, (2.0 - rinv) * (2.0 - rinv)) * (
        1.0 / (BOUND * BOUND))
    sigma = 25.0 * jnp.exp(-2.0 * s2)
    a = sigma * STEP
    alpha = 1.0 - jnp.exp(-a)
    l1a = jnp.log(1.0 - alpha)
    q = l1a
    sh = 1
    while sh < NUM:
        q = q + jnp.concatenate(
            [jnp.zeros((q.shape[0], sh), jnp.float32), q[:, :NUM - sh]], axis=1)
        sh *= 2
    q = q - l1a
    t = jnp.log(alpha) + q
    unmasked = (ci % 4) != 0
    t = jnp.where(jnp.isfinite(t), t, NEG_BIG)
    t_cmp = jnp.where(unmasked, t, NEG_BIG - 0.0)
    in_a = unmasked & (ci < 256)

    ranks_ref[...] = jnp.full((ranks_ref.shape[0], 256), 1000, jnp.int32)
    ones_col = jnp.ones((NUM, 1), jnp.float32)
    for cc in CAND_COLS:
        tc = t[:, cc:cc + 1]
        gt = t_cmp > tc
        tie = (t_cmp == tc) & in_a & (ci < cc)
        pred = jnp.where(gt | tie, 1.0, 0.0)
        cnt = jax.lax.dot_general(
            pred, ones_col, (((1,), (0,)), ((), ())),
            preferred_element_type=jnp.float32)
        ranks_ref[:, cc:cc + 1] = cnt.astype(jnp.int32)


def _tc_ranks(rays_o, rays_d):
    n = rays_o.shape[0]
    grid = n // R_BLOCK
    return pl.pallas_call(
        _tc_ranks_body,
        grid=(grid,),
        in_specs=[
            pl.BlockSpec((R_BLOCK, 3), lambda i: (i, 0)),
            pl.BlockSpec((R_BLOCK, 3), lambda i: (i, 0)),
        ],
        out_specs=pl.BlockSpec((R_BLOCK, 256), lambda i: (i, 0)),
        out_shape=jax.ShapeDtypeStruct((n, 256), jnp.int32),
    )(rays_o, rays_d)


def _sc_compact(ranks):
    import numpy as np
    from jax.experimental.pallas import tpu_sc as plsc

    n = ranks.shape[0]
    info = plsc.get_sparse_core_info()
    nw = info.num_cores * info.num_subcores
    nc = info.num_cores
    rays_per_w = n // nw
    ch = 16
    n_chunks = rays_per_w // ch
    mesh = plsc.VectorSubcoreMesh(core_axis_name="c", subcore_axis_name="s")

    @functools.partial(
        pl.kernel,
        mesh=mesh,
        compiler_params=pltpu.CompilerParams(needs_layout_passes=False),
        out_type=jax.ShapeDtypeStruct((n, INNER), jnp.float32),
        scratch_types=[
            pltpu.VMEM((ch, 256), jnp.int32),
            pltpu.VMEM((ch, INNER), jnp.float32),
            pltpu.VMEM((576,), jnp.int32),
            pltpu.VMEM((576,), jnp.int32),
        ],
    )
    def sc_kernel(ranks_hbm, out_hbm, in_v, out_v, sel_v, g_v):
        wid = lax.axis_index("s") * nc + lax.axis_index("c")
        i16 = lax.iota(jnp.int32, 16)
        zero16 = i16 * 0
        ones16 = zero16 + 1

        def chunk_body(ci, _):
            base = wid * rays_per_w + ci * ch
            pltpu.sync_copy(ranks_hbm.at[pl.ds(base, ch)], in_v)

            def ray_body(r, __):
                rfull = zero16 + r
                for s in range(36):
                    sel_v[pl.ds(16 * s, 16)] = zero16
                for gch in range(16):
                    rv = in_v[r, pl.ds(16 * gch, 16)]
                    plsc.store_scatter(sel_v, [rv], ones16, mask=rv < 576)
                carry = jnp.int32(0)
                for s in range(36):
                    v = sel_v[pl.ds(16 * s, 16)]
                    cs = plsc.cumsum(v) + carry
                    g_v[pl.ds(16 * s, 16)] = cs
                    carry = carry + jnp.sum(v)
                    jj = i16 + (16 * s)
                    q3 = jj // 3
                    colj = q3 + jj + 1
                    slot = q3 + cs
                    val = LO + STEP * colj.astype(jnp.float32)
                    plsc.store_scatter(out_v, [rfull, slot], val, mask=v > 0)
                for mm in range(12):
                    m = i16 + (16 * mm)
                    gidx = jnp.maximum(3 * m - 1, 0)
                    gv = plsc.load_gather(g_v, [gidx])
                    if mm == 0:
                        gv = jnp.where(i16 == 0, 0, gv)
                    slot = m + gv
                    val = LO + STEP * 4.0 * m.astype(jnp.float32)
                    plsc.store_scatter(out_v, [rfull, slot], val)
                return __

            lax.fori_loop(0, ch, ray_body, 0)
            pltpu.sync_copy(out_v, out_hbm.at[pl.ds(base, ch)])
            return _

        lax.fori_loop(0, n_chunks, chunk_body, 0)

    return sc_kernel(ranks)


def _backhalf_jnp(ranks):
    n = ranks.shape[0]
    r = ranks[:, jnp.asarray(CAND_COLS)]
    jidx = jnp.arange(576)
    sel = jnp.any(r[:, :, None] == jidx[None, None, :], axis=1)
    g = jnp.cumsum(sel.astype(jnp.int32), axis=1)
    colj = 4 * (jidx // 3) + (jidx % 3) + 1
    slot_u = jidx // 3 + g
    val_u = (LO + STEP * colj).astype(jnp.float32)
    midx = jnp.arange(K)
    gprev = jnp.where(midx == 0, 0, jnp.take_along_axis(
        g, jnp.maximum(3 * midx - 1, 0)[None, :].repeat(n, 0), axis=1))
    slot_m = midx[None, :] + gprev
    val_m = (LO + STEP * 4.0 * midx).astype(jnp.float32)
    out = jnp.zeros((n, INNER), jnp.float32)
    out = out.at[jnp.arange(n)[:, None], slot_m].set(
        jnp.broadcast_to(val_m[None, :], (n, K)))
    out = out.at[jnp.arange(n)[:, None], jnp.where(sel, slot_u, INNER)].set(
        jnp.broadcast_to(val_u[None, :], (n, 576)), mode="drop")
    return out


def kernel(rays_o, rays_d, n_samples):
    n = rays_o.shape[0]
    nsl = 4
    sl = n // nsl
    outs = []
    for i in range(nsl):
        ranks = _tc_ranks(rays_o[i * sl:(i + 1) * sl],
                          rays_d[i * sl:(i + 1) * sl])
        outs.append(_sc_compact(ranks))
    return jnp.concatenate(outs, axis=0)


# speedup vs baseline: 1.6522x; 1.0329x over previous
import functools
import math

import jax
import jax.numpy as jnp
from jax import lax
from jax.experimental import pallas as pl
from jax.experimental.pallas import tpu as pltpu

NUM = 768
K = 192
INNER = 384
BOUND = 1.125
LO = math.log10(0.05)
HI = math.log10(1.0) - (math.log10(1.0) - math.log10(0.05)) / NUM
STEP = (HI - LO) / (NUM - 1)
LN10 = math.log(10.0)
NEG_BIG = -3.0e38

CAND_COLS = tuple(c for c in range(256) if c % 4 != 0)

R_BLOCK = 128


def _tc_ranks_body(o_ref, d_ref, ranks_ref):
    ci = lax.broadcasted_iota(jnp.int32, (1, NUM), 1)
    c = ci.astype(jnp.float32)
    zlog = LO + STEP * c
    z = jnp.exp(zlog * LN10)
    ox = o_ref[:, 0:1]
    oy = o_ref[:, 1:2]
    oz = o_ref[:, 2:3]
    dx = d_ref[:, 0:1]
    dy = d_ref[:, 1:2]
    dz = d_ref[:, 2:3]
    x = ox + z * dx
    y = oy + z * dy
    w = oz + z * dz
    m2 = x * x + y * y + w * w
    rinv = lax.rsqrt(m2)
    s2 = jnp.where(m2 <= 1.0, m2, (2.0 - rinv) * (2.0 - rinv)) * (
        1.0 / (BOUND * BOUND))
    sigma = 25.0 * jnp.exp(-2.0 * s2)
    a = sigma * STEP
    alpha = 1.0 - jnp.exp(-a)
    l1a = jnp.log(1.0 - alpha)
    q = l1a
    sh = 1
    while sh < NUM:
        q = q + jnp.concatenate(
            [jnp.zeros((q.shape[0], sh), jnp.float32), q[:, :NUM - sh]], axis=1)
        sh *= 2
    q = q - l1a
    t = jnp.log(alpha) + q
    unmasked = (ci % 4) != 0
    t = jnp.where(jnp.isfinite(t), t, NEG_BIG)
    t_cmp = jnp.where(unmasked, t, NEG_BIG - 0.0)
    in_a = unmasked & (ci < 256)

    ranks_ref[...] = jnp.full((ranks_ref.shape[0], 256), 1000, jnp.int32)
    ones_col = jnp.ones((NUM, 1), jnp.float32)
    for cc in CAND_COLS:
        tc = t[:, cc:cc + 1]
        gt = t_cmp > tc
        tie = (t_cmp == tc) & in_a & (ci < cc)
        pred = jnp.where(gt | tie, 1.0, 0.0)
        cnt = jax.lax.dot_general(
            pred, ones_col, (((1,), (0,)), ((), ())),
            preferred_element_type=jnp.float32)
        ranks_ref[:, cc:cc + 1] = cnt.astype(jnp.int32)


def _tc_ranks(rays_o, rays_d):
    n = rays_o.shape[0]
    grid = n // R_BLOCK
    return pl.pallas_call(
        _tc_ranks_body,
        grid=(grid,),
        in_specs=[
            pl.BlockSpec((R_BLOCK, 3), lambda i: (i, 0)),
            pl.BlockSpec((R_BLOCK, 3), lambda i: (i, 0)),
        ],
        out_specs=pl.BlockSpec((R_BLOCK, 256), lambda i: (i, 0)),
        out_shape=jax.ShapeDtypeStruct((n, 256), jnp.int32),
    )(rays_o, rays_d)


def _sc_compact(ranks):
    import numpy as np
    from jax.experimental.pallas import tpu_sc as plsc

    n = ranks.shape[0]
    info = plsc.get_sparse_core_info()
    nw = info.num_cores * info.num_subcores
    nc = info.num_cores
    rays_per_w = n // nw
    ch = 16
    n_chunks = rays_per_w // ch
    mesh = plsc.VectorSubcoreMesh(core_axis_name="c", subcore_axis_name="s")

    @functools.partial(
        pl.kernel,
        mesh=mesh,
        compiler_params=pltpu.CompilerParams(needs_layout_passes=False),
        out_type=jax.ShapeDtypeStruct((n, INNER), jnp.float32),
        scratch_types=[
            pltpu.VMEM((ch, 256), jnp.int32),
            pltpu.VMEM((ch, INNER), jnp.float32),
            pltpu.VMEM((576,), jnp.int32),
            pltpu.VMEM((576,), jnp.int32),
        ],
    )
    def sc_kernel(ranks_hbm, out_hbm, in_v, out_v, sel_v, g_v):
        wid = lax.axis_index("s") * nc + lax.axis_index("c")
        i16 = lax.iota(jnp.int32, 16)
        zero16 = i16 * 0
        ones16 = zero16 + 1

        def chunk_body(ci, _):
            base = wid * rays_per_w + ci * ch
            pltpu.sync_copy(ranks_hbm.at[pl.ds(base, ch)], in_v)

            def ray_body(r, __):
                rfull = zero16 + r
                for s in range(36):
                    sel_v[pl.ds(16 * s, 16)] = zero16
                for gch in range(16):
                    rv = in_v[r, pl.ds(16 * gch, 16)]
                    plsc.store_scatter(sel_v, [rv], ones16, mask=rv < 576)
                carry = jnp.int32(0)
                for s in range(36):
                    v = sel_v[pl.ds(16 * s, 16)]
                    cs = plsc.cumsum(v) + carry
                    g_v[pl.ds(16 * s, 16)] = cs
                    carry = carry + jnp.sum(v)
                    jj = i16 + (16 * s)
                    q3 = jj // 3
                    colj = q3 + jj + 1
                    slot = q3 + cs
                    val = LO + STEP * colj.astype(jnp.float32)
                    plsc.store_scatter(out_v, [rfull, slot], val, mask=v > 0)
                for mm in range(12):
                    m = i16 + (16 * mm)
                    gidx = jnp.maximum(3 * m - 1, 0)
                    gv = plsc.load_gather(g_v, [gidx])
                    if mm == 0:
                        gv = jnp.where(i16 == 0, 0, gv)
                    slot = m + gv
                    val = LO + STEP * 4.0 * m.astype(jnp.float32)
                    plsc.store_scatter(out_v, [rfull, slot], val)
                return __

            lax.fori_loop(0, ch, ray_body, 0)
            pltpu.sync_copy(out_v, out_hbm.at[pl.ds(base, ch)])
            return _

        lax.fori_loop(0, n_chunks, chunk_body, 0)

    return sc_kernel(ranks)


def _backhalf_jnp(ranks):
    n = ranks.shape[0]
    r = ranks[:, jnp.asarray(CAND_COLS)]
    jidx = jnp.arange(576)
    sel = jnp.any(r[:, :, None] == jidx[None, None, :], axis=1)
    g = jnp.cumsum(sel.astype(jnp.int32), axis=1)
    colj = 4 * (jidx // 3) + (jidx % 3) + 1
    slot_u = jidx // 3 + g
    val_u = (LO + STEP * colj).astype(jnp.float32)
    midx = jnp.arange(K)
    gprev = jnp.where(midx == 0, 0, jnp.take_along_axis(
        g, jnp.maximum(3 * midx - 1, 0)[None, :].repeat(n, 0), axis=1))
    slot_m = midx[None, :] + gprev
    val_m = (LO + STEP * 4.0 * midx).astype(jnp.float32)
    out = jnp.zeros((n, INNER), jnp.float32)
    out = out.at[jnp.arange(n)[:, None], slot_m].set(
        jnp.broadcast_to(val_m[None, :], (n, K)))
    out = out.at[jnp.arange(n)[:, None], jnp.where(sel, slot_u, INNER)].set(
        jnp.broadcast_to(val_u[None, :], (n, 576)), mode="drop")
    return out


def kernel(rays_o, rays_d, n_samples):
    n = rays_o.shape[0]
    nsl = 8
    sl = n // nsl
    outs = []
    for i in range(nsl):
        ranks = _tc_ranks(rays_o[i * sl:(i + 1) * sl],
                          rays_d[i * sl:(i + 1) * sl])
        outs.append(_sc_compact(ranks))
    return jnp.concatenate(outs, axis=0)
